# baseline (speedup 1.0000x reference)
import functools

import jax
import jax.numpy as jnp
from jax import lax
from jax.experimental import pallas as pl
from jax.experimental.pallas import tpu as pltpu
from jax.experimental.pallas import tpu_sc as plsc

NC = 19
NBINS = NC * NC
BINS_PAD = 512
B, H, W = 8, 512, 512
ROWS = 256
NW = 32
LANES = 16
CHUNKS = 4
BC = B // CHUNKS
SLABS = NW // BC
RW = H // SLABS


def _argmax_label_body(logits_ref, mask_ref, label_ref):
    best = logits_ref[0, 0]
    idx = jnp.zeros(best.shape, jnp.int32)
    for c in range(1, NC):
        v = logits_ref[0, c]
        m = v > best
        best = jnp.where(m, v, best)
        idx = jnp.where(m, jnp.int32(c), idx)
    label_ref[0] = mask_ref[0] * NC + idx


def _argmax_label(logits, mask, b0):
    grid = (BC, H // ROWS)
    return pl.pallas_call(
        _argmax_label_body,
        grid=grid,
        in_specs=[
            pl.BlockSpec((1, NC, ROWS, W), lambda b, r, b0=b0: (b + b0, 0, r, 0)),
            pl.BlockSpec((1, ROWS, W), lambda b, r, b0=b0: (b + b0, r, 0)),
        ],
        out_specs=pl.BlockSpec((1, ROWS, W), lambda b, r: (b, r, 0)),
        out_shape=jax.ShapeDtypeStruct((BC, H, W), jnp.int32),
    )(logits, mask)


def _sc_hist_body(labels_hbm, out_hbm, lab_v, hist_v, red_v, dma_sem):
    wid = lax.axis_index("s") * 2 + lax.axis_index("c")
    lane_base = lax.iota(jnp.int32, LANES) * BINS_PAD
    ones = jnp.ones((LANES,), jnp.float32)
    zeros = jnp.zeros((LANES,), jnp.float32)

    b = wid // SLABS
    q = wid % SLABS
    cp = pltpu.make_async_copy(
        labels_hbm.at[b, pl.ds(q * RW, RW)], lab_v, dma_sem)
    cp.start()

    def zero_body(j, _):
        hist_v[pl.ds(j * LANES, LANES)] = zeros
        return 0
    lax.fori_loop(0, LANES * BINS_PAD // LANES, zero_body, 0)

    cp.wait()

    def hist_body(r, _):
        for c in range(W // LANES):
            lbl = lab_v[r, pl.ds(c * LANES, LANES)]
            plsc.addupdate_scatter(hist_v, [lane_base + lbl], ones)
        return 0
    lax.fori_loop(0, RW, hist_body, 0)

    for j in range(BINS_PAD // LANES):
        acc = hist_v[pl.ds(j * LANES, LANES)]
        for l in range(1, LANES):
            acc = acc + hist_v[pl.ds(l * BINS_PAD + j * LANES, LANES)]
        red_v[pl.ds(j * LANES, LANES)] = acc

    pltpu.sync_copy(red_v, out_hbm.at[wid])


def _sc_hist(labels):
    mesh = plsc.VectorSubcoreMesh(core_axis_name="c", subcore_axis_name="s")
    return pl.kernel(
        _sc_hist_body,
        out_type=jax.ShapeDtypeStruct((NW, BINS_PAD), jnp.float32),
        mesh=mesh,
        compiler_params=pltpu.CompilerParams(needs_layout_passes=False),
        scratch_types=[
            pltpu.VMEM((RW, W), jnp.int32),
            pltpu.VMEM((LANES * BINS_PAD,), jnp.float32),
            pltpu.VMEM((BINS_PAD,), jnp.float32),
            pltpu.SemaphoreType.DMA,
        ],
    )(labels)


def _stats_body(*refs):
    part_refs, (miou_ref, fwiou_ref) = refs[:CHUNKS], refs[CHUNKS:]
    s = part_refs[0][...]
    for p in part_refs[1:]:
        s = s + p[...]
    s = jnp.sum(s, axis=0, keepdims=True)
    rows = [s[:, i * NC:(i + 1) * NC] for i in range(NC)]
    cm = jnp.concatenate(rows, axis=0)

    ii = lax.broadcasted_iota(jnp.int32, (NC, NC), 0)
    jj = lax.broadcasted_iota(jnp.int32, (NC, NC), 1)
    eye = (ii == jj).astype(jnp.float32)

    diag = jnp.sum(cm * eye, axis=1, keepdims=True)
    rowsum = jnp.sum(cm, axis=1, keepdims=True)
    ones_col = jnp.ones((NC, 1), jnp.float32)
    colsum = lax.dot_general(cm, ones_col, (((0,), (0,)), ((), ())))

    denom = rowsum + colsum - diag
    iu = jnp.where(denom > 0, diag / jnp.where(denom > 0, denom, 1.0), 0.0)
    miou_ref[...] = (jnp.sum(iu) / NC).reshape(1, 1)

    total = jnp.sum(rowsum)
    freq = rowsum / jnp.where(total > 0, total, 1.0)
    fwiou_ref[...] = jnp.sum(jnp.where(freq > 0, freq * iu, 0.0)).reshape(1, 1)


def _stats(parts):
    return pl.pallas_call(
        _stats_body,
        out_shape=(
            jax.ShapeDtypeStruct((1, 1), jnp.float32),
            jax.ShapeDtypeStruct((1, 1), jnp.float32),
        ),
    )(*parts)


def kernel(logits, mask):
    labels = [_argmax_label(logits, mask, k * BC) for k in range(CHUNKS)]
    parts = [_sc_hist(lab) for lab in labels]
    miou, fwiou = _stats(parts)
    return (miou[0, 0], fwiou[0, 0])


# speedup vs baseline: 1.0395x; 1.0395x over previous
import functools

import jax
import jax.numpy as jnp
from jax import lax
from jax.experimental import pallas as pl
from jax.experimental.pallas import tpu as pltpu
from jax.experimental.pallas import tpu_sc as plsc

NC = 19
NBINS = NC * NC
BINS_PAD = 512
B, H, W = 8, 512, 512
ROWS = 256
NW = 32
LANES = 16
CHUNKS = 2
BC = B // CHUNKS
SLABS = NW // BC
RW = H // SLABS


def _argmax_label_body(logits_ref, mask_ref, label_ref):
    best = logits_ref[0, 0]
    idx = jnp.zeros(best.shape, jnp.int32)
    for c in range(1, NC):
        v = logits_ref[0, c]
        m = v > best
        best = jnp.where(m, v, best)
        idx = jnp.where(m, jnp.int32(c), idx)
    col = lax.broadcasted_iota(jnp.int32, idx.shape, 1)
    lane_off = (col & (LANES - 1)) * BINS_PAD
    label_ref[0] = mask_ref[0] * NC + idx + lane_off


def _argmax_label(logits, mask, b0):
    grid = (BC, H // ROWS)
    return pl.pallas_call(
        _argmax_label_body,
        grid=grid,
        in_specs=[
            pl.BlockSpec((1, NC, ROWS, W), lambda b, r, b0=b0: (b + b0, 0, r, 0)),
            pl.BlockSpec((1, ROWS, W), lambda b, r, b0=b0: (b + b0, r, 0)),
        ],
        out_specs=pl.BlockSpec((1, ROWS, W), lambda b, r: (b, r, 0)),
        out_shape=jax.ShapeDtypeStruct((BC, H, W), jnp.int32),
    )(logits, mask)


def _sc_hist_body(labels_hbm, out_hbm, lab_v, hist_v, red_v, dma_sem):
    wid = lax.axis_index("s") * 2 + lax.axis_index("c")
    ones = jnp.ones((LANES,), jnp.float32)
    zeros = jnp.zeros((LANES,), jnp.float32)

    b = wid // SLABS
    q = wid % SLABS
    cp = pltpu.make_async_copy(
        labels_hbm.at[b, pl.ds(q * RW, RW)], lab_v, dma_sem)
    cp.start()

    def zero_body(j, _):
        hist_v[pl.ds(j * LANES, LANES)] = zeros
        return 0
    lax.fori_loop(0, LANES * BINS_PAD // LANES, zero_body, 0)

    cp.wait()

    def hist_body(r, _):
        for c in range(W // LANES):
            lbl = lab_v[r, pl.ds(c * LANES, LANES)]
            plsc.addupdate_scatter(hist_v, [lbl], ones)
        return 0
    lax.fori_loop(0, RW, hist_body, 0)

    for j in range(BINS_PAD // LANES):
        acc = hist_v[pl.ds(j * LANES, LANES)]
        for l in range(1, LANES):
            acc = acc + hist_v[pl.ds(l * BINS_PAD + j * LANES, LANES)]
        red_v[pl.ds(j * LANES, LANES)] = acc

    pltpu.sync_copy(red_v, out_hbm.at[wid])


def _sc_hist(labels):
    mesh = plsc.VectorSubcoreMesh(core_axis_name="c", subcore_axis_name="s")
    return pl.kernel(
        _sc_hist_body,
        out_type=jax.ShapeDtypeStruct((NW, BINS_PAD), jnp.float32),
        mesh=mesh,
        compiler_params=pltpu.CompilerParams(needs_layout_passes=False),
        scratch_types=[
            pltpu.VMEM((RW, W), jnp.int32),
            pltpu.VMEM((LANES * BINS_PAD,), jnp.float32),
            pltpu.VMEM((BINS_PAD,), jnp.float32),
            pltpu.SemaphoreType.DMA,
        ],
    )(labels)


def _stats_body(*refs):
    part_refs, (miou_ref, fwiou_ref) = refs[:CHUNKS], refs[CHUNKS:]
    s = part_refs[0][...]
    for p in part_refs[1:]:
        s = s + p[...]
    s = jnp.sum(s, axis=0, keepdims=True)
    rows = [s[:, i * NC:(i + 1) * NC] for i in range(NC)]
    cm = jnp.concatenate(rows, axis=0)

    ii = lax.broadcasted_iota(jnp.int32, (NC, NC), 0)
    jj = lax.broadcasted_iota(jnp.int32, (NC, NC), 1)
    eye = (ii == jj).astype(jnp.float32)

    diag = jnp.sum(cm * eye, axis=1, keepdims=True)
    rowsum = jnp.sum(cm, axis=1, keepdims=True)
    ones_col = jnp.ones((NC, 1), jnp.float32)
    colsum = lax.dot_general(cm, ones_col, (((0,), (0,)), ((), ())))

    denom = rowsum + colsum - diag
    iu = jnp.where(denom > 0, diag / jnp.where(denom > 0, denom, 1.0), 0.0)
    miou_ref[...] = (jnp.sum(iu) / NC).reshape(1, 1)

    total = jnp.sum(rowsum)
    freq = rowsum / jnp.where(total > 0, total, 1.0)
    fwiou_ref[...] = jnp.sum(jnp.where(freq > 0, freq * iu, 0.0)).reshape(1, 1)


def _stats(parts):
    return pl.pallas_call(
        _stats_body,
        out_shape=(
            jax.ShapeDtypeStruct((1, 1), jnp.float32),
            jax.ShapeDtypeStruct((1, 1), jnp.float32),
        ),
    )(*parts)


def kernel(logits, mask):
    labels = [_argmax_label(logits, mask, k * BC) for k in range(CHUNKS)]
    parts = [_sc_hist(lab) for lab in labels]
    miou, fwiou = _stats(parts)
    return (miou[0, 0], fwiou[0, 0])


# speedup vs baseline: 1.0403x; 1.0008x over previous
import functools

import jax
import jax.numpy as jnp
from jax import lax
from jax.experimental import pallas as pl
from jax.experimental.pallas import tpu as pltpu
from jax.experimental.pallas import tpu_sc as plsc

NC = 19
NBINS = NC * NC
BINS_PAD = 512
B, H, W = 8, 512, 512
ROWS = 256
NW = 32
LANES = 16
LANE_STRIDE = BINS_PAD + 1
CHUNKS = 2
BC = B // CHUNKS
SLABS = NW // BC
RW = H // SLABS


def _argmax_label_body(logits_ref, mask_ref, label_ref):
    best = logits_ref[0, 0]
    idx = jnp.zeros(best.shape, jnp.int32)
    for c in range(1, NC):
        v = logits_ref[0, c]
        m = v > best
        best = jnp.where(m, v, best)
        idx = jnp.where(m, jnp.int32(c), idx)
    col = lax.broadcasted_iota(jnp.int32, idx.shape, 1)
    lane_off = (col & (LANES - 1)) * LANE_STRIDE
    label_ref[0] = mask_ref[0] * NC + idx + lane_off


def _argmax_label(logits, mask, b0):
    grid = (BC, H // ROWS)
    return pl.pallas_call(
        _argmax_label_body,
        grid=grid,
        in_specs=[
            pl.BlockSpec((1, NC, ROWS, W), lambda b, r, b0=b0: (b + b0, 0, r, 0)),
            pl.BlockSpec((1, ROWS, W), lambda b, r, b0=b0: (b + b0, r, 0)),
        ],
        out_specs=pl.BlockSpec((1, ROWS, W), lambda b, r: (b, r, 0)),
        out_shape=jax.ShapeDtypeStruct((BC, H, W), jnp.int32),
    )(logits, mask)


def _sc_hist_body(labels_hbm, out_hbm, lab_v, hist_v, red_v, dma_sem):
    wid = lax.axis_index("s") * 2 + lax.axis_index("c")
    ones = jnp.ones((LANES,), jnp.float32)
    zeros = jnp.zeros((LANES,), jnp.float32)

    b = wid // SLABS
    q = wid % SLABS
    cp = pltpu.make_async_copy(
        labels_hbm.at[b, pl.ds(q * RW, RW)], lab_v, dma_sem)
    cp.start()

    def zero_body(j, _):
        hist_v[pl.ds(j * LANES, LANES)] = zeros
        return 0
    lax.fori_loop(0, LANES * LANE_STRIDE // LANES + 1, zero_body, 0)

    cp.wait()

    def hist_body(r, _):
        for c in range(W // LANES):
            lbl = lab_v[r, pl.ds(c * LANES, LANES)]
            plsc.addupdate_scatter(hist_v, [lbl], ones)
        return 0
    lax.fori_loop(0, RW, hist_body, 0)

    for j in range(BINS_PAD // LANES):
        acc = hist_v[pl.ds(j * LANES, LANES)]
        for l in range(1, LANES):
            acc = acc + hist_v[pl.ds(l * LANE_STRIDE + j * LANES, LANES)]
        red_v[pl.ds(j * LANES, LANES)] = acc

    pltpu.sync_copy(red_v, out_hbm.at[wid])


def _sc_hist(labels):
    mesh = plsc.VectorSubcoreMesh(core_axis_name="c", subcore_axis_name="s")
    return pl.kernel(
        _sc_hist_body,
        out_type=jax.ShapeDtypeStruct((NW, BINS_PAD), jnp.float32),
        mesh=mesh,
        compiler_params=pltpu.CompilerParams(needs_layout_passes=False),
        scratch_types=[
            pltpu.VMEM((RW, W), jnp.int32),
            pltpu.VMEM((LANES * LANE_STRIDE + LANES,), jnp.float32),
            pltpu.VMEM((BINS_PAD,), jnp.float32),
            pltpu.SemaphoreType.DMA,
        ],
    )(labels)


def _stats_body(*refs):
    part_refs, (miou_ref, fwiou_ref) = refs[:CHUNKS], refs[CHUNKS:]
    s = part_refs[0][...]
    for p in part_refs[1:]:
        s = s + p[...]
    s = jnp.sum(s, axis=0, keepdims=True)
    rows = [s[:, i * NC:(i + 1) * NC] for i in range(NC)]
    cm = jnp.concatenate(rows, axis=0)

    ii = lax.broadcasted_iota(jnp.int32, (NC, NC), 0)
    jj = lax.broadcasted_iota(jnp.int32, (NC, NC), 1)
    eye = (ii == jj).astype(jnp.float32)

    diag = jnp.sum(cm * eye, axis=1, keepdims=True)
    rowsum = jnp.sum(cm, axis=1, keepdims=True)
    ones_col = jnp.ones((NC, 1), jnp.float32)
    colsum = lax.dot_general(cm, ones_col, (((0,), (0,)), ((), ())))

    denom = rowsum + colsum - diag
    iu = jnp.where(denom > 0, diag / jnp.where(denom > 0, denom, 1.0), 0.0)
    miou_ref[...] = (jnp.sum(iu) / NC).reshape(1, 1)

    total = jnp.sum(rowsum)
    freq = rowsum / jnp.where(total > 0, total, 1.0)
    fwiou_ref[...] = jnp.sum(jnp.where(freq > 0, freq * iu, 0.0)).reshape(1, 1)


def _stats(parts):
    return pl.pallas_call(
        _stats_body,
        out_shape=(
            jax.ShapeDtypeStruct((1, 1), jnp.float32),
            jax.ShapeDtypeStruct((1, 1), jnp.float32),
        ),
    )(*parts)


def kernel(logits, mask):
    labels = [_argmax_label(logits, mask, k * BC) for k in range(CHUNKS)]
    parts = [_sc_hist(lab) for lab in labels]
    miou, fwiou = _stats(parts)
    return (miou[0, 0], fwiou[0, 0])
